# baseline (speedup 1.0000x reference)
import functools

import jax
import jax.numpy as jnp
from jax import lax
from jax.experimental import pallas as pl
from jax.experimental.pallas import tpu as pltpu
from jax.experimental.pallas import tpu_sc as plsc

_EMBED = 64
_BATCH = 16384
_NC = 2
_NS = 16
_NW = _NC * _NS
_BPW = _BATCH // _NW
_NCHUNK = 4
_CH = _BPW // _NCHUNK
_LANES = 16


def _sc_body(target_hbm, context_hbm, win_hbm, wout_hbm, out_hbm,
             idx_t, idx_c, rows_a, rows_c, out_v, sem):
    wid = lax.axis_index("s") * _NC + lax.axis_index("c")
    base = wid * _BPW

    pltpu.sync_copy(target_hbm.at[pl.ds(base, _BPW)], idx_t)
    pltpu.sync_copy(context_hbm.at[pl.ds(base, _BPW)], idx_c)

    copies = []
    for j in range(_NCHUNK):
        copies.append(pltpu.async_copy(
            win_hbm.at[idx_t.at[pl.ds(j * _CH, _CH)]],
            rows_a.at[pl.ds(j * _CH, _CH)], sem))
        copies.append(pltpu.async_copy(
            wout_hbm.at[idx_c.at[pl.ds(j * _CH, _CH)]],
            rows_c.at[pl.ds(j * _CH, _CH)], sem))
    for cp in copies:
        cp.wait()

    lanes = lax.iota(jnp.int32, _LANES)

    def group_body(g, carry):
        row0 = g * _LANES
        out_vec = jnp.zeros((_LANES,), jnp.float32)
        for i in range(_LANES):
            r = row0 + i
            p = jnp.zeros((_LANES,), jnp.float32)
            for c in range(_EMBED // _LANES):
                a = rows_a[r, pl.ds(c * _LANES, _LANES)]
                b = rows_c[r, pl.ds(c * _LANES, _LANES)]
                p = p + a * b
            s = jnp.sum(p)
            out_vec = jnp.where(lanes == i, s, out_vec)
        out_v[pl.ds(row0, _LANES)] = out_vec
        return carry

    lax.fori_loop(0, _BPW // _LANES, group_body, 0)

    pltpu.sync_copy(out_v, out_hbm.at[pl.ds(base, _BPW)])


def kernel(target, context, W_in, W_out):
    run = functools.partial(
        pl.kernel,
        out_type=jax.ShapeDtypeStruct((_BATCH,), jnp.float32),
        mesh=plsc.VectorSubcoreMesh(core_axis_name="c", subcore_axis_name="s"),
        compiler_params=pltpu.CompilerParams(
            needs_layout_passes=False, use_tc_tiling_on_sc=False),
        scratch_types=[
            pltpu.VMEM((_BPW,), jnp.int32),
            pltpu.VMEM((_BPW,), jnp.int32),
            pltpu.VMEM((_BPW, _EMBED), jnp.float32),
            pltpu.VMEM((_BPW, _EMBED), jnp.float32),
            pltpu.VMEM((_BPW,), jnp.float32),
            pltpu.SemaphoreType.DMA,
        ],
    )(_sc_body)
    return run(target.astype(jnp.int32), context.astype(jnp.int32),
               W_in, W_out)
